# baseline (speedup 1.0000x reference)
import jax
import jax.numpy as jnp
from jax import lax
from jax.experimental import pallas as pl

_HIDDEN = 2048
_EXPERTS = 64
_TM = 512


def _router_logits_kernel(x_ref, w_ref, o_ref):
    o_ref[...] = lax.dot_general(
        x_ref[...],
        w_ref[...],
        dimension_numbers=(((1,), (0,)), ((), ())),
        preferred_element_type=jnp.float32,
    )


def kernel(hidden_states, weight):
    hs2 = hidden_states.reshape(-1, 2 * _HIDDEN)
    m2 = hs2.shape[0]
    wt = weight.astype(jnp.float32).T
    wbig = jnp.zeros((2 * _HIDDEN, 2 * _EXPERTS), jnp.float32)
    wbig = lax.dynamic_update_slice(wbig, wt, (0, 0))
    wbig = lax.dynamic_update_slice(wbig, wt, (_HIDDEN, _EXPERTS))
    out = pl.pallas_call(
        _router_logits_kernel,
        grid=(m2 // _TM,),
        in_specs=[
            pl.BlockSpec((_TM, 2 * _HIDDEN), lambda i: (i, 0)),
            pl.BlockSpec((2 * _HIDDEN, 2 * _EXPERTS), lambda i: (0, 0)),
        ],
        out_specs=pl.BlockSpec((_TM, 2 * _EXPERTS), lambda i: (i, 0)),
        out_shape=jax.ShapeDtypeStruct((m2, 2 * _EXPERTS), jnp.float32),
    )(hs2, wbig)
    return out.reshape(-1, _EXPERTS)


# speedup vs baseline: 4.0048x; 4.0048x over previous
import jax
import jax.numpy as jnp
from jax import lax
from jax.experimental import pallas as pl

_HIDDEN = 2048
_EXPERTS = 64
_TM = 512


def _router_logits_kernel(x_ref, w_ref, o_ref):
    o_ref[...] = lax.dot_general(
        w_ref[...],
        x_ref[...],
        dimension_numbers=(((1,), (1,)), ((), ())),
        preferred_element_type=jnp.float32,
    )


def kernel(hidden_states, weight):
    hs = hidden_states.reshape(-1, _HIDDEN)
    m = hs.shape[0]
    grid = (m // _TM,)
    out_t = pl.pallas_call(
        _router_logits_kernel,
        grid=grid,
        in_specs=[
            pl.BlockSpec((_TM, _HIDDEN), lambda i: (i, 0)),
            pl.BlockSpec((_EXPERTS, _HIDDEN), lambda i: (0, 0)),
        ],
        out_specs=pl.BlockSpec((_EXPERTS, _TM), lambda i: (0, i)),
        out_shape=jax.ShapeDtypeStruct((_EXPERTS, m), jnp.float32),
    )(hs, weight)
    return out_t.T
